# Initial kernel scaffold; baseline (speedup 1.0000x reference)
import jax
import jax.numpy as jnp
from jax.experimental import pallas as pl


def kernel(x, lens, table):
    raise NotImplementedError("write your pallas kernel here")


# speedup vs baseline: 5.0117x; 5.0117x over previous
import functools

import jax
import jax.numpy as jnp
from jax import lax
from jax.experimental import pallas as pl
from jax.experimental.pallas import tpu as pltpu
from jax.experimental.pallas import tpu_sc as plsc

N_PAD = 256
WINDOW = 128
MASK_BLK = 512


def _make_mask_kernel(vocab, batch, seq):
    def body(x_ref, lens_ref, out_ref):
        b_blk = x_ref.shape[0]
        col = lax.broadcasted_iota(jnp.int32, (b_blk, seq), 1)
        row = lax.broadcasted_iota(jnp.int32, (b_blk, seq), 0)
        pad_idx = vocab + ((row * seq + col) % N_PAD)
        out_ref[...] = jnp.where(col < lens_ref[...], x_ref[...], pad_idx)

    grid = (batch // MASK_BLK,)
    return pl.pallas_call(
        body,
        grid=grid,
        in_specs=[
            pl.BlockSpec((MASK_BLK, seq), lambda i: (i, 0)),
            pl.BlockSpec((MASK_BLK, 1), lambda i: (i, 0)),
        ],
        out_specs=pl.BlockSpec((MASK_BLK, seq), lambda i: (i, 0)),
        out_shape=jax.ShapeDtypeStruct((batch, seq), jnp.int32),
    )


def _make_sc_gather(n_idx, vocab_pad, dim):
    mesh = plsc.VectorSubcoreMesh(core_axis_name="c", subcore_axis_name="s")

    @functools.partial(
        pl.kernel,
        out_type=jax.ShapeDtypeStruct((n_idx, dim), jnp.float32),
        mesh=mesh,
    )
    def sc_kernel(table_hbm, idx_hbm, out_hbm):
        def body(i_vmem, o_vmem):
            pltpu.sync_copy(table_hbm.at[i_vmem.at[0]], o_vmem)

        pltpu.emit_pipeline(
            body,
            grid=(n_idx // WINDOW,),
            in_specs=[pl.BlockSpec((1, WINDOW), lambda i: (0, i))],
            out_specs=[pl.BlockSpec((WINDOW, dim), lambda i: (i, 0))],
            core_axis_name=("c", "s"),
            dimension_semantics=(pltpu.PARALLEL,),
        )(idx_hbm, out_hbm)

    return sc_kernel


@jax.jit
def kernel(x, lens, table):
    batch, seq = x.shape
    vocab, dim = table.shape
    x = x.astype(jnp.int32)
    lens = lens.astype(jnp.int32).reshape(batch, 1)

    masked_idx = _make_mask_kernel(vocab, batch, seq)(x, lens)

    table_pad = jnp.concatenate(
        [table, jnp.zeros((N_PAD, dim), table.dtype)], axis=0
    )

    n_idx = batch * seq
    out = _make_sc_gather(n_idx, vocab + N_PAD, dim)(
        table_pad, masked_idx.reshape(1, n_idx)
    )
    return out.reshape(batch, seq, dim)


# speedup vs baseline: 5.1219x; 1.0220x over previous
import functools

import jax
import jax.numpy as jnp
from jax import lax
from jax.experimental import pallas as pl
from jax.experimental.pallas import tpu as pltpu
from jax.experimental.pallas import tpu_sc as plsc

N_PAD = 256
WINDOW = 128
MASK_BLK = 512


def _make_mask_kernel(vocab, batch, seq):
    def body(x_ref, lens_ref, out_ref):
        b_blk = x_ref.shape[0]
        col = lax.broadcasted_iota(jnp.int32, (b_blk, seq), 1)
        row = lax.broadcasted_iota(jnp.int32, (b_blk, seq), 0)
        pad_idx = vocab + ((row * seq + col) % N_PAD)
        out_ref[...] = jnp.where(col < lens_ref[...], x_ref[...], pad_idx)

    grid = (batch // MASK_BLK,)
    return pl.pallas_call(
        body,
        grid=grid,
        in_specs=[
            pl.BlockSpec((MASK_BLK, seq), lambda i: (i, 0)),
            pl.BlockSpec((MASK_BLK, 1), lambda i: (i, 0)),
        ],
        out_specs=pl.BlockSpec((MASK_BLK, seq), lambda i: (i, 0)),
        out_shape=jax.ShapeDtypeStruct((batch, seq), jnp.int32),
    )


def _make_sc_gather(n_idx, vocab_pad, dim):
    mesh = plsc.VectorSubcoreMesh(core_axis_name="c", subcore_axis_name="s")
    n_rows = n_idx // WINDOW

    @functools.partial(
        pl.kernel,
        out_type=jax.ShapeDtypeStruct((n_idx, dim), jnp.float32),
        mesh=mesh,
        scratch_types=[pltpu.SemaphoreType.DMA],
    )
    def sc_kernel(table_hbm, idx_hbm, out_hbm, sem):
        def body(i_vmem, o_vmem):
            cp0 = pltpu.async_copy(
                table_hbm.at[i_vmem.at[0]], o_vmem.at[pl.ds(0, WINDOW)], sem
            )
            cp1 = pltpu.async_copy(
                table_hbm.at[i_vmem.at[1]], o_vmem.at[pl.ds(WINDOW, WINDOW)], sem
            )
            cp0.wait()
            cp1.wait()

        pltpu.emit_pipeline(
            body,
            grid=(n_rows // 2,),
            in_specs=[pl.BlockSpec((2, WINDOW), lambda i: (i, 0))],
            out_specs=[pl.BlockSpec((2 * WINDOW, dim), lambda i: (i, 0))],
            core_axis_name=("c", "s"),
            dimension_semantics=(pltpu.PARALLEL,),
        )(idx_hbm, out_hbm)

    return sc_kernel


@jax.jit
def kernel(x, lens, table):
    batch, seq = x.shape
    vocab, dim = table.shape
    x = x.astype(jnp.int32)
    lens = lens.astype(jnp.int32).reshape(batch, 1)

    masked_idx = _make_mask_kernel(vocab, batch, seq)(x, lens)

    table_pad = jnp.concatenate(
        [table, jnp.zeros((N_PAD, dim), table.dtype)], axis=0
    )

    n_idx = batch * seq
    out = _make_sc_gather(n_idx, vocab + N_PAD, dim)(
        table_pad, masked_idx.reshape(n_idx // WINDOW, WINDOW)
    )
    return out.reshape(batch, seq, dim)
